# baseline (speedup 1.0000x reference)
import functools

import jax
import jax.numpy as jnp
from jax import lax
from jax.experimental import pallas as pl
from jax.experimental.pallas import tpu as pltpu
from jax.experimental.pallas import tpu_sc as plsc

N = 10000
D = 128
E = 320000
EH = E // 2
NCORE = 2
NSUB = 16
EPT = EH // NSUB
CHUNK = 128
NCHUNK = 80
EPTP = NCHUNK * CHUNK
NP = 10240
NPT = NP // NSUB
PADIDX = NP - 1
GRP = 20
NGRP = NCHUNK // GRP


def _rsqrt_newton(d):
    seed = jnp.where(d < 64.0, 1.0 / 8.0,
                     jnp.where(d < 4096.0, 1.0 / 64.0, 1.0 / 512.0))
    y = seed.astype(jnp.float32)
    for _ in range(11):
        y = y * (1.5 - 0.5 * d * y * y)
    return jnp.where(d > 0.5, y, 0.0)


@functools.partial(
    pl.kernel,
    mesh=plsc.VectorSubcoreMesh(core_axis_name="c", subcore_axis_name="s"),
    out_type=jax.ShapeDtypeStruct((NCORE, NP), jnp.float32),
    compiler_params=pltpu.CompilerParams(needs_layout_passes=False),
    scratch_types=[
        pltpu.VMEM((NCHUNK, CHUNK), jnp.int32),
        pltpu.VMEM((EPTP,), jnp.int32),
        pltpu.VMEM((EPTP,), jnp.float32),
        pltpu.VMEM((NP,), jnp.float32),
        pltpu.VMEM((NPT,), jnp.float32),
        pltpu.VMEM((NPT,), jnp.float32),
        pltpu.VMEM_SHARED((NP,), jnp.float32),
        pltpu.VMEM_SHARED((NP,), jnp.float32),
        pltpu.VMEM_SHARED((NP,), jnp.float32),
        pltpu.SemaphoreType.DMA,
        pltpu.SemaphoreType.DMA,
    ],
)
def _sc_coeffs(row_hbm, col_hbm, out_hbm, row_buf, col_buf, vals_buf,
               dinv_full, slice_buf, dinv_slice, sp_deg, sp_t, sp_dinv,
               sem_in, sem_scat):
    cid = lax.axis_index("c")
    sid = lax.axis_index("s")
    nbase = sid * NPT

    pltpu.async_copy(row_hbm.at[cid, sid], row_buf, sem_in)
    pltpu.async_copy(col_hbm.at[cid, sid], col_buf, sem_in)

    zeros = jnp.zeros((16,), jnp.float32)
    ones = jnp.ones((16,), jnp.float32)

    def zbody(i, c):
        slice_buf[pl.ds(i * 16, 16)] = zeros
        return c
    lax.fori_loop(0, NPT // 16, zbody, 0)
    pltpu.sync_copy(slice_buf, sp_deg.at[pl.ds(nbase, NPT)])
    pltpu.sync_copy(slice_buf, sp_t.at[pl.ds(nbase, NPT)])

    def obody(i, c):
        vals_buf[pl.ds(i * 16, 16)] = ones
        return c
    lax.fori_loop(0, EPTP // 16, obody, 0)

    pltpu.make_async_copy(row_hbm.at[cid, sid], row_buf, sem_in).wait()
    pltpu.make_async_copy(col_hbm.at[cid, sid], col_buf, sem_in).wait()

    plsc.subcore_barrier()

    def _scatter_all(target):
        def grp_body(g, c):
            def start_body(j, c2):
                k = g * GRP + j
                pltpu.async_copy(vals_buf.at[pl.ds(k * CHUNK, CHUNK)],
                                 target.at[row_buf.at[k]], sem_scat, add=True)
                return c2
            lax.fori_loop(0, GRP, start_body, 0)

            def drain_body(j, c2):
                k = g * GRP + j
                pltpu.make_async_copy(vals_buf.at[pl.ds(k * CHUNK, CHUNK)],
                                      target.at[row_buf.at[k]], sem_scat).wait()
                return c2
            lax.fori_loop(0, GRP, drain_body, 0)
            return c
        lax.fori_loop(0, NGRP, grp_body, 0)

    _scatter_all(sp_deg)

    plsc.subcore_barrier()

    pltpu.sync_copy(sp_deg.at[pl.ds(nbase, NPT)], slice_buf)

    def dbody(i, c):
        d = slice_buf[pl.ds(i * 16, 16)]
        dinv_slice[pl.ds(i * 16, 16)] = _rsqrt_newton(d)
        return c
    lax.fori_loop(0, NPT // 16, dbody, 0)
    pltpu.sync_copy(dinv_slice, sp_dinv.at[pl.ds(nbase, NPT)])

    plsc.subcore_barrier()

    pltpu.sync_copy(sp_dinv, dinv_full)

    def gbody(i, c):
        cidx = col_buf[pl.ds(i * 16, 16)]
        vals_buf[pl.ds(i * 16, 16)] = plsc.load_gather(dinv_full, [cidx])
        return c
    lax.fori_loop(0, EPTP // 16, gbody, 0)

    _scatter_all(sp_t)

    plsc.subcore_barrier()

    pltpu.sync_copy(sp_t.at[pl.ds(nbase, NPT)], slice_buf)

    def fbody(i, c):
        t = slice_buf[pl.ds(i * 16, 16)]
        dv = dinv_slice[pl.ds(i * 16, 16)]
        slice_buf[pl.ds(i * 16, 16)] = t * dv
        return c
    lax.fori_loop(0, NPT // 16, fbody, 0)
    pltpu.sync_copy(slice_buf, out_hbm.at[cid, pl.ds(nbase, NPT)])


def _mm_body(x_ref, win_ref, wout_ref, wloop_ref, a_in_ref, a_out_ref,
             a_loop_ref):
    x = x_ref[...]
    a_in_ref[...] = jnp.dot(x, win_ref[...], preferred_element_type=jnp.float32)
    a_out_ref[...] = jnp.dot(x, wout_ref[...], preferred_element_type=jnp.float32)
    a_loop_ref[...] = jnp.dot(x, wloop_ref[...], preferred_element_type=jnp.float32)


def _combine_body(a_in_ref, a_out_ref, a_loop_ref, sin_ref, sout_ref,
                  g_ref, b_ref, o_ref):
    pre = (a_in_ref[...] * sin_ref[...] + a_out_ref[...] * sout_ref[...]
           + a_loop_ref[...]) * jnp.float32(1.0 / 3.0)
    mean = jnp.mean(pre, axis=0, keepdims=True)
    var = jnp.mean(pre * pre, axis=0, keepdims=True) - mean * mean
    inv = lax.rsqrt(var + 1e-5)
    o_ref[...] = jnp.tanh(g_ref[...] * (pre - mean) * inv + b_ref[...])


def kernel(batch, x, edge_index, rel_embed, W_in, W_out, W_loop, gamma, beta):
    ei4 = edge_index.reshape(2, NCORE, NSUB, EPT)
    eip = jnp.pad(ei4, ((0, 0), (0, 0), (0, 0), (0, EPTP - EPT)),
                  constant_values=PADIDX)
    row_t = eip[0].reshape(NCORE, NSUB, NCHUNK, CHUNK)
    col_t = eip[1]

    s2 = _sc_coeffs(row_t, col_t)

    fdt = jnp.float32
    a_in, a_out, a_loop = pl.pallas_call(
        _mm_body,
        out_shape=(jax.ShapeDtypeStruct((N, D), fdt),
                   jax.ShapeDtypeStruct((N, D), fdt),
                   jax.ShapeDtypeStruct((N, D), fdt)),
    )(x, W_in, W_out, W_loop)

    sin = s2[0, :N].reshape(N, 1)
    sout = s2[1, :N].reshape(N, 1)

    out = pl.pallas_call(
        _combine_body,
        out_shape=jax.ShapeDtypeStruct((N, D), fdt),
    )(a_in, a_out, a_loop, sin, sout, gamma.reshape(1, D), beta.reshape(1, D))
    return out, rel_embed


# speedup vs baseline: 1.1503x; 1.1503x over previous
import functools

import jax
import jax.numpy as jnp
from jax import lax
from jax.experimental import pallas as pl
from jax.experimental.pallas import tpu as pltpu
from jax.experimental.pallas import tpu_sc as plsc

N = 10000
D = 128
E = 320000
EH = E // 2
NCORE = 2
NSUB = 16
EPT = EH // NSUB
CHUNK = 128
NCHUNK = 80
EPTP = NCHUNK * CHUNK
NP = 10240
NPT = NP // NSUB
PADIDX = NP - 1
GRP = 20
NGRP = NCHUNK // GRP


def _rsqrt_newton(d):
    seed = jnp.where(d < 64.0, 1.0 / 8.0,
                     jnp.where(d < 4096.0, 1.0 / 64.0, 1.0 / 512.0))
    y = seed.astype(jnp.float32)
    for _ in range(11):
        y = y * (1.5 - 0.5 * d * y * y)
    return jnp.where(d > 0.5, y, 0.0)


@functools.partial(
    pl.kernel,
    mesh=plsc.VectorSubcoreMesh(core_axis_name="c", subcore_axis_name="s"),
    out_type=jax.ShapeDtypeStruct((NCORE, NP), jnp.float32),
    compiler_params=pltpu.CompilerParams(needs_layout_passes=False),
    scratch_types=[
        pltpu.VMEM((NCHUNK, CHUNK), jnp.int32),
        pltpu.VMEM((EPTP,), jnp.int32),
        pltpu.VMEM((EPTP,), jnp.float32),
        pltpu.VMEM((NP,), jnp.float32),
        pltpu.VMEM((NPT,), jnp.float32),
        pltpu.VMEM((NPT,), jnp.float32),
        pltpu.VMEM_SHARED((NP,), jnp.float32),
        pltpu.VMEM_SHARED((NP,), jnp.float32),
        pltpu.VMEM_SHARED((NP,), jnp.float32),
        pltpu.SemaphoreType.DMA,
        pltpu.SemaphoreType.DMA,
    ],
)
def _sc_coeffs(row_hbm, col_hbm, out_hbm, row_buf, col_buf, vals_buf,
               dinv_full, slice_buf, dinv_slice, sp_deg, sp_t, sp_dinv,
               sem_in, sem_scat):
    cid = lax.axis_index("c")
    sid = lax.axis_index("s")
    nbase = sid * NPT

    pltpu.async_copy(row_hbm.at[cid, sid], row_buf, sem_in)
    pltpu.async_copy(col_hbm.at[cid, sid], col_buf, sem_in)

    zeros = jnp.zeros((16,), jnp.float32)
    ones = jnp.ones((16,), jnp.float32)

    def zbody(i, c):
        slice_buf[pl.ds(i * 16, 16)] = zeros
        return c
    lax.fori_loop(0, NPT // 16, zbody, 0)
    pltpu.sync_copy(slice_buf, sp_deg.at[pl.ds(nbase, NPT)])
    pltpu.sync_copy(slice_buf, sp_t.at[pl.ds(nbase, NPT)])

    def obody(i, c):
        vals_buf[pl.ds(i * 16, 16)] = ones
        return c
    lax.fori_loop(0, EPTP // 16, obody, 0)

    pltpu.make_async_copy(row_hbm.at[cid, sid], row_buf, sem_in).wait()
    pltpu.make_async_copy(col_hbm.at[cid, sid], col_buf, sem_in).wait()

    plsc.subcore_barrier()

    def _scatter_all(target):
        def grp_body(g, c):
            def start_body(j, c2):
                k = g * GRP + j
                pltpu.async_copy(vals_buf.at[pl.ds(k * CHUNK, CHUNK)],
                                 target.at[row_buf.at[k]], sem_scat, add=True)
                return c2
            lax.fori_loop(0, GRP, start_body, 0)

            def drain_body(j, c2):
                k = g * GRP + j
                pltpu.make_async_copy(vals_buf.at[pl.ds(k * CHUNK, CHUNK)],
                                      target.at[row_buf.at[k]], sem_scat).wait()
                return c2
            lax.fori_loop(0, GRP, drain_body, 0)
            return c
        lax.fori_loop(0, NGRP, grp_body, 0)

    _scatter_all(sp_deg)

    plsc.subcore_barrier()

    pltpu.sync_copy(sp_deg.at[pl.ds(nbase, NPT)], slice_buf)

    def dbody(i, c):
        d = slice_buf[pl.ds(i * 16, 16)]
        dinv_slice[pl.ds(i * 16, 16)] = _rsqrt_newton(d)
        return c
    lax.fori_loop(0, NPT // 16, dbody, 0)
    pltpu.sync_copy(dinv_slice, sp_dinv.at[pl.ds(nbase, NPT)])

    plsc.subcore_barrier()

    pltpu.sync_copy(sp_dinv, dinv_full)

    def gbody(i, c):
        cidx = col_buf[pl.ds(i * 16, 16)]
        vals_buf[pl.ds(i * 16, 16)] = plsc.load_gather(dinv_full, [cidx])
        return c
    lax.fori_loop(0, EPTP // 16, gbody, 0)

    _scatter_all(sp_t)

    plsc.subcore_barrier()

    pltpu.sync_copy(sp_t.at[pl.ds(nbase, NPT)], slice_buf)

    def fbody(i, c):
        t = slice_buf[pl.ds(i * 16, 16)]
        dv = dinv_slice[pl.ds(i * 16, 16)]
        slice_buf[pl.ds(i * 16, 16)] = t * dv
        return c
    lax.fori_loop(0, NPT // 16, fbody, 0)
    pltpu.sync_copy(slice_buf, out_hbm.at[cid, pl.ds(nbase, NPT)])


def _mm_body(x_ref, win_ref, wout_ref, wloop_ref, a_in_ref, a_out_ref,
             a_loop_ref):
    x = x_ref[...]
    a_in_ref[...] = jnp.dot(
        x, win_ref[...], preferred_element_type=jnp.float32).astype(jnp.bfloat16)
    a_out_ref[...] = jnp.dot(
        x, wout_ref[...], preferred_element_type=jnp.float32).astype(jnp.bfloat16)
    a_loop_ref[...] = jnp.dot(
        x, wloop_ref[...], preferred_element_type=jnp.float32).astype(jnp.bfloat16)


def _combine_body(a_in_ref, a_out_ref, a_loop_ref, s_ref, g_ref, b_ref,
                  o_ref):
    sin = jnp.transpose(s_ref[0:1, :N])
    sout = jnp.transpose(s_ref[1:2, :N])
    pre = (a_in_ref[...].astype(jnp.float32) * sin
           + a_out_ref[...].astype(jnp.float32) * sout
           + a_loop_ref[...].astype(jnp.float32)) * jnp.float32(1.0 / 3.0)
    mean = jnp.mean(pre, axis=0, keepdims=True)
    var = jnp.mean(pre * pre, axis=0, keepdims=True) - mean * mean
    inv = lax.rsqrt(var + 1e-5)
    o_ref[...] = jnp.tanh(g_ref[...] * (pre - mean) * inv + b_ref[...])


def kernel(batch, x, edge_index, rel_embed, W_in, W_out, W_loop, gamma, beta):
    ei4 = edge_index.reshape(2, NCORE, NSUB, EPT)
    eip = jnp.pad(ei4, ((0, 0), (0, 0), (0, 0), (0, EPTP - EPT)),
                  constant_values=PADIDX)
    row_t = eip[0].reshape(NCORE, NSUB, NCHUNK, CHUNK)
    col_t = eip[1]

    s2 = _sc_coeffs(row_t, col_t)

    bdt = jnp.bfloat16
    a_in, a_out, a_loop = pl.pallas_call(
        _mm_body,
        out_shape=(jax.ShapeDtypeStruct((N, D), bdt),
                   jax.ShapeDtypeStruct((N, D), bdt),
                   jax.ShapeDtypeStruct((N, D), bdt)),
    )(x, W_in, W_out, W_loop)

    out = pl.pallas_call(
        _combine_body,
        out_shape=jax.ShapeDtypeStruct((N, D), jnp.float32),
    )(a_in, a_out, a_loop, s2, gamma.reshape(1, D), beta.reshape(1, D))
    return out, rel_embed


# speedup vs baseline: 1.1836x; 1.0290x over previous
import functools

import jax
import jax.numpy as jnp
from jax import lax
from jax.experimental import pallas as pl
from jax.experimental.pallas import tpu as pltpu
from jax.experimental.pallas import tpu_sc as plsc

N = 10000
D = 128
E = 320000
EH = E // 2
NCORE = 2
NSUB = 16
CHUNK = 128
NCHT = EH // CHUNK
NCHUNK = 80
EPTP = NCHUNK * CHUNK
BCH = 72
NP = 10240
NPT = NP // NSUB
GRP = 24
NGRP = BCH // GRP


def _rsqrt_newton(d):
    seed = jnp.where(d < 64.0, 1.0 / 8.0,
                     jnp.where(d < 4096.0, 1.0 / 64.0, 1.0 / 512.0))
    y = seed.astype(jnp.float32)
    for _ in range(11):
        y = y * (1.5 - 0.5 * d * y * y)
    return jnp.where(d > 0.5, y, 0.0)


@functools.partial(
    pl.kernel,
    mesh=plsc.VectorSubcoreMesh(core_axis_name="c", subcore_axis_name="s"),
    out_type=jax.ShapeDtypeStruct((NCORE, NP), jnp.float32),
    compiler_params=pltpu.CompilerParams(needs_layout_passes=False),
    scratch_types=[
        pltpu.VMEM((NCHUNK, CHUNK), jnp.int32),
        pltpu.VMEM((EPTP,), jnp.int32),
        pltpu.VMEM((EPTP,), jnp.float32),
        pltpu.VMEM((NP,), jnp.float32),
        pltpu.VMEM((NPT,), jnp.float32),
        pltpu.VMEM((NPT,), jnp.float32),
        pltpu.VMEM_SHARED((NP,), jnp.float32),
        pltpu.VMEM_SHARED((NP,), jnp.float32),
        pltpu.VMEM_SHARED((NP,), jnp.float32),
        pltpu.SemaphoreType.DMA,
        pltpu.SemaphoreType.DMA,
    ],
)
def _sc_coeffs(row_hbm, col_hbm, out_hbm, row_buf, col_buf, vals_buf,
               dinv_full, slice_buf, dinv_slice, sp_deg, sp_t, sp_dinv,
               sem_in, sem_scat):
    cid = lax.axis_index("c")
    sid = lax.axis_index("s")
    nbase = sid * NPT
    lo12 = sid < 12
    is15 = sid == 15
    cbase = pl.multiple_of(
        jnp.where(lo12, NCHUNK * sid, 12 * NCHUNK + BCH * (sid - 12)), 8)

    izeros = jnp.zeros((16,), jnp.int32)

    @pl.when(sid >= 12)
    def _():
        for j in range(((NCHUNK - BCH - 2) * CHUNK) // 16):
            col_buf[pl.ds((BCH + 2) * CHUNK + j * 16, 16)] = izeros

    @pl.when(jnp.logical_and(sid >= 12, sid < 15))
    def _():
        for j in range((2 * CHUNK) // 16):
            col_buf[pl.ds(BCH * CHUNK + j * 16, 16)] = izeros

    pltpu.async_copy(row_hbm.at[cid, pl.ds(cbase, BCH)],
                     row_buf.at[pl.ds(0, BCH)], sem_in)
    pltpu.async_copy(col_hbm.at[cid, pl.ds(cbase * CHUNK, BCH * CHUNK)],
                     col_buf.at[pl.ds(0, BCH * CHUNK)], sem_in)

    @pl.when(lo12)
    def _():
        pltpu.async_copy(row_hbm.at[cid, pl.ds(cbase + BCH, NCHUNK - BCH)],
                         row_buf.at[pl.ds(BCH, NCHUNK - BCH)], sem_in)
        pltpu.async_copy(
            col_hbm.at[cid, pl.ds((cbase + BCH) * CHUNK, (NCHUNK - BCH) * CHUNK)],
            col_buf.at[pl.ds(BCH * CHUNK, (NCHUNK - BCH) * CHUNK)], sem_in)

    @pl.when(is15)
    def _():
        pltpu.async_copy(row_hbm.at[cid, NCHT - 2], row_buf.at[BCH], sem_in)
        pltpu.async_copy(row_hbm.at[cid, NCHT - 1], row_buf.at[BCH + 1], sem_in)
        pltpu.async_copy(col_hbm.at[cid, pl.ds((NCHT - 2) * CHUNK, 2 * CHUNK)],
                         col_buf.at[pl.ds(BCH * CHUNK, 2 * CHUNK)], sem_in)

    zeros = jnp.zeros((16,), jnp.float32)
    ones = jnp.ones((16,), jnp.float32)

    def zbody(i, c):
        for j in range(8):
            slice_buf[pl.ds(i * 128 + j * 16, 16)] = zeros
        return c
    lax.fori_loop(0, NPT // 128, zbody, 0)
    pltpu.sync_copy(slice_buf, sp_deg.at[pl.ds(nbase, NPT)])
    pltpu.sync_copy(slice_buf, sp_t.at[pl.ds(nbase, NPT)])

    def obody(i, c):
        for j in range(8):
            vals_buf[pl.ds(i * 128 + j * 16, 16)] = ones
        return c
    lax.fori_loop(0, EPTP // 128, obody, 0)

    pltpu.make_async_copy(row_hbm.at[cid, pl.ds(cbase, BCH)],
                          row_buf.at[pl.ds(0, BCH)], sem_in).wait()
    pltpu.make_async_copy(col_hbm.at[cid, pl.ds(cbase * CHUNK, BCH * CHUNK)],
                          col_buf.at[pl.ds(0, BCH * CHUNK)], sem_in).wait()

    @pl.when(lo12)
    def _():
        pltpu.make_async_copy(row_hbm.at[cid, pl.ds(cbase + BCH, NCHUNK - BCH)],
                              row_buf.at[pl.ds(BCH, NCHUNK - BCH)], sem_in).wait()
        pltpu.make_async_copy(
            col_hbm.at[cid, pl.ds((cbase + BCH) * CHUNK, (NCHUNK - BCH) * CHUNK)],
            col_buf.at[pl.ds(BCH * CHUNK, (NCHUNK - BCH) * CHUNK)], sem_in).wait()

    @pl.when(is15)
    def _():
        pltpu.make_async_copy(row_hbm.at[cid, NCHT - 2],
                              row_buf.at[BCH], sem_in).wait()
        pltpu.make_async_copy(row_hbm.at[cid, NCHT - 1],
                              row_buf.at[BCH + 1], sem_in).wait()
        pltpu.make_async_copy(col_hbm.at[cid, pl.ds((NCHT - 2) * CHUNK, 2 * CHUNK)],
                              col_buf.at[pl.ds(BCH * CHUNK, 2 * CHUNK)], sem_in).wait()

    plsc.subcore_barrier()

    def _scatter_all(target):
        def grp_body(g, c):
            def start_body(j, c2):
                k = g * GRP + j
                pltpu.async_copy(vals_buf.at[pl.ds(k * CHUNK, CHUNK)],
                                 target.at[row_buf.at[k]], sem_scat, add=True)
                return c2
            lax.fori_loop(0, GRP, start_body, 0)

            def drain_body(j, c2):
                k = g * GRP + j
                pltpu.make_async_copy(vals_buf.at[pl.ds(k * CHUNK, CHUNK)],
                                      target.at[row_buf.at[k]], sem_scat).wait()
                return c2
            lax.fori_loop(0, GRP, drain_body, 0)
            return c
        lax.fori_loop(0, NGRP, grp_body, 0)

        @pl.when(lo12)
        def _():
            def s8(j, c2):
                k = BCH + j
                pltpu.async_copy(vals_buf.at[pl.ds(k * CHUNK, CHUNK)],
                                 target.at[row_buf.at[k]], sem_scat, add=True)
                return c2
            lax.fori_loop(0, NCHUNK - BCH, s8, 0)

            def d8(j, c2):
                k = BCH + j
                pltpu.make_async_copy(vals_buf.at[pl.ds(k * CHUNK, CHUNK)],
                                      target.at[row_buf.at[k]], sem_scat).wait()
                return c2
            lax.fori_loop(0, NCHUNK - BCH, d8, 0)

        @pl.when(is15)
        def _():
            pltpu.sync_copy(vals_buf.at[pl.ds(BCH * CHUNK, CHUNK)],
                            target.at[row_buf.at[BCH]], add=True)
            pltpu.sync_copy(vals_buf.at[pl.ds((BCH + 1) * CHUNK, CHUNK)],
                            target.at[row_buf.at[BCH + 1]], add=True)

    _scatter_all(sp_deg)

    plsc.subcore_barrier()

    pltpu.sync_copy(sp_deg.at[pl.ds(nbase, NPT)], slice_buf)

    def dbody(i, c):
        for j in range(2):
            d = slice_buf[pl.ds(i * 32 + j * 16, 16)]
            dinv_slice[pl.ds(i * 32 + j * 16, 16)] = _rsqrt_newton(d)
        return c
    lax.fori_loop(0, NPT // 32, dbody, 0)
    pltpu.sync_copy(dinv_slice, sp_dinv.at[pl.ds(nbase, NPT)])

    plsc.subcore_barrier()

    pltpu.sync_copy(sp_dinv, dinv_full)

    def gbody(i, c):
        for j in range(4):
            cidx = col_buf[pl.ds(i * 64 + j * 16, 16)]
            vals_buf[pl.ds(i * 64 + j * 16, 16)] = plsc.load_gather(
                dinv_full, [cidx])
        return c
    lax.fori_loop(0, EPTP // 64, gbody, 0)

    _scatter_all(sp_t)

    plsc.subcore_barrier()

    pltpu.sync_copy(sp_t.at[pl.ds(nbase, NPT)], slice_buf)

    def fbody(i, c):
        for j in range(8):
            t = slice_buf[pl.ds(i * 128 + j * 16, 16)]
            dv = dinv_slice[pl.ds(i * 128 + j * 16, 16)]
            slice_buf[pl.ds(i * 128 + j * 16, 16)] = t * dv
        return c
    lax.fori_loop(0, NPT // 128, fbody, 0)
    pltpu.sync_copy(slice_buf, out_hbm.at[cid, pl.ds(nbase, NPT)])


def _mm_body(x_ref, win_ref, wout_ref, wloop_ref, a_in_ref, a_out_ref,
             a_loop_ref):
    x = x_ref[...]
    a_in_ref[...] = jnp.dot(
        x, win_ref[...], preferred_element_type=jnp.float32).astype(jnp.bfloat16)
    a_out_ref[...] = jnp.dot(
        x, wout_ref[...], preferred_element_type=jnp.float32).astype(jnp.bfloat16)
    a_loop_ref[...] = jnp.dot(
        x, wloop_ref[...], preferred_element_type=jnp.float32).astype(jnp.bfloat16)


def _combine_body(a_in_ref, a_out_ref, a_loop_ref, s_ref, g_ref, b_ref,
                  o_ref):
    sin = jnp.transpose(s_ref[0:1, :N])
    sout = jnp.transpose(s_ref[1:2, :N])
    pre = (a_in_ref[...].astype(jnp.float32) * sin
           + a_out_ref[...].astype(jnp.float32) * sout
           + a_loop_ref[...].astype(jnp.float32)) * jnp.float32(1.0 / 3.0)
    mean = jnp.mean(pre, axis=0, keepdims=True)
    var = jnp.mean(pre * pre, axis=0, keepdims=True) - mean * mean
    inv = lax.rsqrt(var + 1e-5)
    o_ref[...] = jnp.tanh(g_ref[...] * (pre - mean) * inv + b_ref[...])


def kernel(batch, x, edge_index, rel_embed, W_in, W_out, W_loop, gamma, beta):
    rows3 = edge_index[0].reshape(NCORE, NCHT, CHUNK)
    cols1 = edge_index[1].reshape(NCORE, EH)

    s2 = _sc_coeffs(rows3, cols1)

    bdt = jnp.bfloat16
    a_in, a_out, a_loop = pl.pallas_call(
        _mm_body,
        out_shape=(jax.ShapeDtypeStruct((N, D), bdt),
                   jax.ShapeDtypeStruct((N, D), bdt),
                   jax.ShapeDtypeStruct((N, D), bdt)),
    )(x, W_in, W_out, W_loop)

    out = pl.pallas_call(
        _combine_body,
        out_shape=jax.ShapeDtypeStruct((N, D), jnp.float32),
    )(a_in, a_out, a_loop, s2, gamma.reshape(1, D), beta.reshape(1, D))
    return out, rel_embed


# speedup vs baseline: 1.3286x; 1.1225x over previous
import functools

import jax
import jax.numpy as jnp
from jax import lax
from jax.experimental import pallas as pl
from jax.experimental.pallas import tpu as pltpu
from jax.experimental.pallas import tpu_sc as plsc

N = 10000
D = 128
E = 320000
EH = E // 2
NCORE = 2
NSUB = 16
CHUNK = 128
NCHT = EH // CHUNK
NCHUNK = 80
EPTP = NCHUNK * CHUNK
BCH = 72
NP = 10240
NPT = NP // NSUB
GRP = 24
NGRP = BCH // GRP


def _rsqrt_newton(d):
    seed = jnp.where(d < 64.0, 1.0 / 8.0,
                     jnp.where(d < 4096.0, 1.0 / 64.0, 1.0 / 512.0))
    y = seed.astype(jnp.float32)
    for _ in range(11):
        y = y * (1.5 - 0.5 * d * y * y)
    return jnp.where(d > 0.5, y, 0.0)


@functools.partial(
    pl.kernel,
    mesh=plsc.VectorSubcoreMesh(core_axis_name="c", subcore_axis_name="s"),
    out_type=jax.ShapeDtypeStruct((NCORE, NP), jnp.float32),
    compiler_params=pltpu.CompilerParams(needs_layout_passes=False),
    scratch_types=[
        pltpu.VMEM((NCHUNK, CHUNK), jnp.int32),
        pltpu.VMEM((EPTP,), jnp.int32),
        pltpu.VMEM((EPTP,), jnp.float32),
        pltpu.VMEM((NP,), jnp.float32),
        pltpu.VMEM((NPT,), jnp.float32),
        pltpu.VMEM((NPT,), jnp.float32),
        pltpu.VMEM_SHARED((NP,), jnp.float32),
        pltpu.VMEM_SHARED((NP,), jnp.float32),
        pltpu.VMEM_SHARED((NP,), jnp.float32),
        pltpu.SemaphoreType.DMA,
        pltpu.SemaphoreType.DMA,
    ],
)
def _sc_coeffs(row_hbm, col_hbm, out_hbm, row_buf, col_buf, vals_buf,
               dinv_full, slice_buf, dinv_slice, sp_deg, sp_t, sp_dinv,
               sem_in, sem_scat):
    cid = lax.axis_index("c")
    sid = lax.axis_index("s")
    nbase = sid * NPT
    lo12 = sid < 12
    is15 = sid == 15
    cbase = pl.multiple_of(
        jnp.where(lo12, NCHUNK * sid, 12 * NCHUNK + BCH * (sid - 12)), 8)

    izeros = jnp.zeros((16,), jnp.int32)

    @pl.when(sid >= 12)
    def _():
        for j in range(((NCHUNK - BCH - 2) * CHUNK) // 16):
            col_buf[pl.ds((BCH + 2) * CHUNK + j * 16, 16)] = izeros

    @pl.when(jnp.logical_and(sid >= 12, sid < 15))
    def _():
        for j in range((2 * CHUNK) // 16):
            col_buf[pl.ds(BCH * CHUNK + j * 16, 16)] = izeros

    pltpu.async_copy(row_hbm.at[0, cid, pl.ds(cbase, BCH)],
                     row_buf.at[pl.ds(0, BCH)], sem_in)
    pltpu.async_copy(col_hbm.at[1, cid, pl.ds(cbase * CHUNK, BCH * CHUNK)],
                     col_buf.at[pl.ds(0, BCH * CHUNK)], sem_in)

    @pl.when(lo12)
    def _():
        pltpu.async_copy(row_hbm.at[0, cid, pl.ds(cbase + BCH, NCHUNK - BCH)],
                         row_buf.at[pl.ds(BCH, NCHUNK - BCH)], sem_in)
        pltpu.async_copy(
            col_hbm.at[1, cid, pl.ds((cbase + BCH) * CHUNK, (NCHUNK - BCH) * CHUNK)],
            col_buf.at[pl.ds(BCH * CHUNK, (NCHUNK - BCH) * CHUNK)], sem_in)

    @pl.when(is15)
    def _():
        pltpu.async_copy(row_hbm.at[0, cid, NCHT - 2], row_buf.at[BCH], sem_in)
        pltpu.async_copy(row_hbm.at[0, cid, NCHT - 1], row_buf.at[BCH + 1], sem_in)
        pltpu.async_copy(col_hbm.at[1, cid, pl.ds((NCHT - 2) * CHUNK, 2 * CHUNK)],
                         col_buf.at[pl.ds(BCH * CHUNK, 2 * CHUNK)], sem_in)

    zeros = jnp.zeros((16,), jnp.float32)
    ones = jnp.ones((16,), jnp.float32)

    def zbody(i, c):
        for j in range(8):
            slice_buf[pl.ds(i * 128 + j * 16, 16)] = zeros
        return c
    lax.fori_loop(0, NPT // 128, zbody, 0)
    pltpu.sync_copy(slice_buf, sp_deg.at[pl.ds(nbase, NPT)])
    pltpu.sync_copy(slice_buf, sp_t.at[pl.ds(nbase, NPT)])

    def obody(i, c):
        for j in range(8):
            vals_buf[pl.ds(i * 128 + j * 16, 16)] = ones
        return c
    lax.fori_loop(0, EPTP // 128, obody, 0)

    pltpu.make_async_copy(row_hbm.at[0, cid, pl.ds(cbase, BCH)],
                          row_buf.at[pl.ds(0, BCH)], sem_in).wait()
    pltpu.make_async_copy(col_hbm.at[1, cid, pl.ds(cbase * CHUNK, BCH * CHUNK)],
                          col_buf.at[pl.ds(0, BCH * CHUNK)], sem_in).wait()

    @pl.when(lo12)
    def _():
        pltpu.make_async_copy(row_hbm.at[0, cid, pl.ds(cbase + BCH, NCHUNK - BCH)],
                              row_buf.at[pl.ds(BCH, NCHUNK - BCH)], sem_in).wait()
        pltpu.make_async_copy(
            col_hbm.at[1, cid, pl.ds((cbase + BCH) * CHUNK, (NCHUNK - BCH) * CHUNK)],
            col_buf.at[pl.ds(BCH * CHUNK, (NCHUNK - BCH) * CHUNK)], sem_in).wait()

    @pl.when(is15)
    def _():
        pltpu.make_async_copy(row_hbm.at[0, cid, NCHT - 2],
                              row_buf.at[BCH], sem_in).wait()
        pltpu.make_async_copy(row_hbm.at[0, cid, NCHT - 1],
                              row_buf.at[BCH + 1], sem_in).wait()
        pltpu.make_async_copy(col_hbm.at[1, cid, pl.ds((NCHT - 2) * CHUNK, 2 * CHUNK)],
                              col_buf.at[pl.ds(BCH * CHUNK, 2 * CHUNK)], sem_in).wait()

    plsc.subcore_barrier()

    def _scatter_all(target):
        def grp_body(g, c):
            def start_body(j, c2):
                k = g * GRP + j
                pltpu.async_copy(vals_buf.at[pl.ds(k * CHUNK, CHUNK)],
                                 target.at[row_buf.at[k]], sem_scat, add=True)
                return c2
            lax.fori_loop(0, GRP, start_body, 0)

            def drain_body(j, c2):
                k = g * GRP + j
                pltpu.make_async_copy(vals_buf.at[pl.ds(k * CHUNK, CHUNK)],
                                      target.at[row_buf.at[k]], sem_scat).wait()
                return c2
            lax.fori_loop(0, GRP, drain_body, 0)
            return c
        lax.fori_loop(0, NGRP, grp_body, 0)

        @pl.when(lo12)
        def _():
            def s8(j, c2):
                k = BCH + j
                pltpu.async_copy(vals_buf.at[pl.ds(k * CHUNK, CHUNK)],
                                 target.at[row_buf.at[k]], sem_scat, add=True)
                return c2
            lax.fori_loop(0, NCHUNK - BCH, s8, 0)

            def d8(j, c2):
                k = BCH + j
                pltpu.make_async_copy(vals_buf.at[pl.ds(k * CHUNK, CHUNK)],
                                      target.at[row_buf.at[k]], sem_scat).wait()
                return c2
            lax.fori_loop(0, NCHUNK - BCH, d8, 0)

        @pl.when(is15)
        def _():
            pltpu.sync_copy(vals_buf.at[pl.ds(BCH * CHUNK, CHUNK)],
                            target.at[row_buf.at[BCH]], add=True)
            pltpu.sync_copy(vals_buf.at[pl.ds((BCH + 1) * CHUNK, CHUNK)],
                            target.at[row_buf.at[BCH + 1]], add=True)

    _scatter_all(sp_deg)

    plsc.subcore_barrier()

    pltpu.sync_copy(sp_deg.at[pl.ds(nbase, NPT)], slice_buf)

    def dbody(i, c):
        for j in range(2):
            d = slice_buf[pl.ds(i * 32 + j * 16, 16)]
            dinv_slice[pl.ds(i * 32 + j * 16, 16)] = _rsqrt_newton(d)
        return c
    lax.fori_loop(0, NPT // 32, dbody, 0)
    pltpu.sync_copy(dinv_slice, sp_dinv.at[pl.ds(nbase, NPT)])

    plsc.subcore_barrier()

    pltpu.sync_copy(sp_dinv, dinv_full)

    def gbody(i, c):
        for j in range(4):
            cidx = col_buf[pl.ds(i * 64 + j * 16, 16)]
            vals_buf[pl.ds(i * 64 + j * 16, 16)] = plsc.load_gather(
                dinv_full, [cidx])
        return c
    lax.fori_loop(0, EPTP // 64, gbody, 0)

    _scatter_all(sp_t)

    plsc.subcore_barrier()

    pltpu.sync_copy(sp_t.at[pl.ds(nbase, NPT)], slice_buf)

    def fbody(i, c):
        for j in range(8):
            t = slice_buf[pl.ds(i * 128 + j * 16, 16)]
            dv = dinv_slice[pl.ds(i * 128 + j * 16, 16)]
            slice_buf[pl.ds(i * 128 + j * 16, 16)] = t * dv
        return c
    lax.fori_loop(0, NPT // 128, fbody, 0)
    pltpu.sync_copy(slice_buf, out_hbm.at[cid, pl.ds(nbase, NPT)])


def _mm_body(x_ref, win_ref, wout_ref, wloop_ref, a_in_ref, a_out_ref,
             a_loop_ref):
    x = x_ref[...]
    a_in_ref[...] = jnp.dot(
        x, win_ref[...], preferred_element_type=jnp.float32).astype(jnp.bfloat16)
    a_out_ref[...] = jnp.dot(
        x, wout_ref[...], preferred_element_type=jnp.float32).astype(jnp.bfloat16)
    a_loop_ref[...] = jnp.dot(
        x, wloop_ref[...], preferred_element_type=jnp.float32).astype(jnp.bfloat16)


def _combine_body(a_in_ref, a_out_ref, a_loop_ref, s_ref, g_ref, b_ref,
                  o_ref):
    sin = jnp.transpose(s_ref[0:1, :N])
    sout = jnp.transpose(s_ref[1:2, :N])
    pre = (a_in_ref[...].astype(jnp.float32) * sin
           + a_out_ref[...].astype(jnp.float32) * sout
           + a_loop_ref[...].astype(jnp.float32)) * jnp.float32(1.0 / 3.0)
    mean = jnp.mean(pre, axis=0, keepdims=True)
    var = jnp.mean(pre * pre, axis=0, keepdims=True) - mean * mean
    inv = lax.rsqrt(var + 1e-5)
    o_ref[...] = jnp.tanh(g_ref[...] * (pre - mean) * inv + b_ref[...])


def kernel(batch, x, edge_index, rel_embed, W_in, W_out, W_loop, gamma, beta):
    ei4 = edge_index.reshape(2, NCORE, NCHT, CHUNK)
    ei2 = edge_index.reshape(2, NCORE, EH)

    s2 = _sc_coeffs(ei4, ei2)

    bdt = jnp.bfloat16
    a_in, a_out, a_loop = pl.pallas_call(
        _mm_body,
        out_shape=(jax.ShapeDtypeStruct((N, D), bdt),
                   jax.ShapeDtypeStruct((N, D), bdt),
                   jax.ShapeDtypeStruct((N, D), bdt)),
    )(x, W_in, W_out, W_loop)

    out = pl.pallas_call(
        _combine_body,
        out_shape=jax.ShapeDtypeStruct((N, D), jnp.float32),
    )(a_in, a_out, a_loop, s2, gamma.reshape(1, D), beta.reshape(1, D))
    return out, rel_embed


# speedup vs baseline: 1.3991x; 1.0531x over previous
import functools

import jax
import jax.numpy as jnp
from jax import lax
from jax.experimental import pallas as pl
from jax.experimental.pallas import tpu as pltpu
from jax.experimental.pallas import tpu_sc as plsc

N = 10000
D = 128
E = 320000
EH = E // 2
NCORE = 2
NSUB = 16
CHUNK = 128
NCHT = EH // CHUNK
NCHUNK = 80
EPTP = NCHUNK * CHUNK
BCH = 72
NP = 10240
NPT = NP // NSUB
GRP = 24
NGRP = BCH // GRP


def _rsqrt_newton(d):
    seed = jnp.where(d < 64.0, 1.0 / 8.0,
                     jnp.where(d < 4096.0, 1.0 / 64.0, 1.0 / 512.0))
    y = seed.astype(jnp.float32)
    for _ in range(11):
        y = y * (1.5 - 0.5 * d * y * y)
    return jnp.where(d > 0.5, y, 0.0)


@functools.partial(
    pl.kernel,
    mesh=plsc.VectorSubcoreMesh(core_axis_name="c", subcore_axis_name="s"),
    out_type=jax.ShapeDtypeStruct((NCORE, NP), jnp.float32),
    compiler_params=pltpu.CompilerParams(needs_layout_passes=False),
    scratch_types=[
        pltpu.VMEM((NCHUNK, CHUNK), jnp.int32),
        pltpu.VMEM((EPTP,), jnp.int32),
        pltpu.VMEM((EPTP,), jnp.float32),
        pltpu.VMEM((NP,), jnp.float32),
        pltpu.VMEM((NPT,), jnp.float32),
        pltpu.VMEM((NPT,), jnp.float32),
        pltpu.VMEM_SHARED((NP,), jnp.float32),
        pltpu.VMEM_SHARED((NP,), jnp.float32),
        pltpu.VMEM_SHARED((NP,), jnp.float32),
        pltpu.SemaphoreType.DMA,
        pltpu.SemaphoreType.DMA,
    ],
)
def _sc_coeffs(row_hbm, col_hbm, out_hbm, row_buf, col_buf, vals_buf,
               dinv_full, slice_buf, dinv_slice, sp_deg, sp_t, sp_dinv,
               sem_in, sem_scat):
    cid = lax.axis_index("c")
    sid = lax.axis_index("s")
    nbase = sid * NPT
    lo12 = sid < 12
    is15 = sid == 15
    cbase = pl.multiple_of(
        jnp.where(lo12, NCHUNK * sid, 12 * NCHUNK + BCH * (sid - 12)), 8)

    izeros = jnp.zeros((16,), jnp.int32)

    @pl.when(sid >= 12)
    def _():
        for j in range(((NCHUNK - BCH - 2) * CHUNK) // 16):
            col_buf[pl.ds((BCH + 2) * CHUNK + j * 16, 16)] = izeros

    @pl.when(jnp.logical_and(sid >= 12, sid < 15))
    def _():
        for j in range((2 * CHUNK) // 16):
            col_buf[pl.ds(BCH * CHUNK + j * 16, 16)] = izeros

    pltpu.async_copy(row_hbm.at[0, cid, pl.ds(cbase, BCH)],
                     row_buf.at[pl.ds(0, BCH)], sem_in)
    pltpu.async_copy(col_hbm.at[1, cid, pl.ds(cbase * CHUNK, BCH * CHUNK)],
                     col_buf.at[pl.ds(0, BCH * CHUNK)], sem_in)

    @pl.when(lo12)
    def _():
        pltpu.async_copy(row_hbm.at[0, cid, pl.ds(cbase + BCH, NCHUNK - BCH)],
                         row_buf.at[pl.ds(BCH, NCHUNK - BCH)], sem_in)
        pltpu.async_copy(
            col_hbm.at[1, cid, pl.ds((cbase + BCH) * CHUNK, (NCHUNK - BCH) * CHUNK)],
            col_buf.at[pl.ds(BCH * CHUNK, (NCHUNK - BCH) * CHUNK)], sem_in)

    @pl.when(is15)
    def _():
        pltpu.async_copy(row_hbm.at[0, cid, NCHT - 2], row_buf.at[BCH], sem_in)
        pltpu.async_copy(row_hbm.at[0, cid, NCHT - 1], row_buf.at[BCH + 1], sem_in)
        pltpu.async_copy(col_hbm.at[1, cid, pl.ds((NCHT - 2) * CHUNK, 2 * CHUNK)],
                         col_buf.at[pl.ds(BCH * CHUNK, 2 * CHUNK)], sem_in)

    zeros = jnp.zeros((16,), jnp.float32)
    ones = jnp.ones((16,), jnp.float32)

    def zbody(i, c):
        for j in range(8):
            slice_buf[pl.ds(i * 128 + j * 16, 16)] = zeros
        return c
    lax.fori_loop(0, NPT // 128, zbody, 0)
    pltpu.sync_copy(slice_buf, sp_deg.at[pl.ds(nbase, NPT)])
    pltpu.sync_copy(slice_buf, sp_t.at[pl.ds(nbase, NPT)])

    def obody(i, c):
        for j in range(8):
            vals_buf[pl.ds(i * 128 + j * 16, 16)] = ones
        return c
    lax.fori_loop(0, EPTP // 128, obody, 0)

    pltpu.make_async_copy(row_hbm.at[0, cid, pl.ds(cbase, BCH)],
                          row_buf.at[pl.ds(0, BCH)], sem_in).wait()
    pltpu.make_async_copy(col_hbm.at[1, cid, pl.ds(cbase * CHUNK, BCH * CHUNK)],
                          col_buf.at[pl.ds(0, BCH * CHUNK)], sem_in).wait()

    @pl.when(lo12)
    def _():
        pltpu.make_async_copy(row_hbm.at[0, cid, pl.ds(cbase + BCH, NCHUNK - BCH)],
                              row_buf.at[pl.ds(BCH, NCHUNK - BCH)], sem_in).wait()
        pltpu.make_async_copy(
            col_hbm.at[1, cid, pl.ds((cbase + BCH) * CHUNK, (NCHUNK - BCH) * CHUNK)],
            col_buf.at[pl.ds(BCH * CHUNK, (NCHUNK - BCH) * CHUNK)], sem_in).wait()

    @pl.when(is15)
    def _():
        pltpu.make_async_copy(row_hbm.at[0, cid, NCHT - 2],
                              row_buf.at[BCH], sem_in).wait()
        pltpu.make_async_copy(row_hbm.at[0, cid, NCHT - 1],
                              row_buf.at[BCH + 1], sem_in).wait()
        pltpu.make_async_copy(col_hbm.at[1, cid, pl.ds((NCHT - 2) * CHUNK, 2 * CHUNK)],
                              col_buf.at[pl.ds(BCH * CHUNK, 2 * CHUNK)], sem_in).wait()

    plsc.subcore_barrier()

    def _scatter_all(target):
        def grp_body(g, c):
            def start_body(j, c2):
                k = g * GRP + j
                pltpu.async_copy(vals_buf.at[pl.ds(k * CHUNK, CHUNK)],
                                 target.at[row_buf.at[k]], sem_scat, add=True)
                return c2
            lax.fori_loop(0, GRP, start_body, 0)

            def drain_body(j, c2):
                k = g * GRP + j
                pltpu.make_async_copy(vals_buf.at[pl.ds(k * CHUNK, CHUNK)],
                                      target.at[row_buf.at[k]], sem_scat).wait()
                return c2
            lax.fori_loop(0, GRP, drain_body, 0)
            return c
        lax.fori_loop(0, NGRP, grp_body, 0)

        @pl.when(lo12)
        def _():
            def s8(j, c2):
                k = BCH + j
                pltpu.async_copy(vals_buf.at[pl.ds(k * CHUNK, CHUNK)],
                                 target.at[row_buf.at[k]], sem_scat, add=True)
                return c2
            lax.fori_loop(0, NCHUNK - BCH, s8, 0)

            def d8(j, c2):
                k = BCH + j
                pltpu.make_async_copy(vals_buf.at[pl.ds(k * CHUNK, CHUNK)],
                                      target.at[row_buf.at[k]], sem_scat).wait()
                return c2
            lax.fori_loop(0, NCHUNK - BCH, d8, 0)

        @pl.when(is15)
        def _():
            pltpu.sync_copy(vals_buf.at[pl.ds(BCH * CHUNK, CHUNK)],
                            target.at[row_buf.at[BCH]], add=True)
            pltpu.sync_copy(vals_buf.at[pl.ds((BCH + 1) * CHUNK, CHUNK)],
                            target.at[row_buf.at[BCH + 1]], add=True)

    _scatter_all(sp_deg)

    plsc.subcore_barrier()

    pltpu.sync_copy(sp_deg.at[pl.ds(nbase, NPT)], slice_buf)

    def dbody(i, c):
        for j in range(2):
            d = slice_buf[pl.ds(i * 32 + j * 16, 16)]
            dinv_slice[pl.ds(i * 32 + j * 16, 16)] = _rsqrt_newton(d)
        return c
    lax.fori_loop(0, NPT // 32, dbody, 0)
    pltpu.sync_copy(dinv_slice, sp_dinv.at[pl.ds(nbase, NPT)])

    plsc.subcore_barrier()

    pltpu.sync_copy(sp_dinv, dinv_full)

    def gbody(i, c):
        for j in range(4):
            cidx = col_buf[pl.ds(i * 64 + j * 16, 16)]
            vals_buf[pl.ds(i * 64 + j * 16, 16)] = plsc.load_gather(
                dinv_full, [cidx])
        return c
    lax.fori_loop(0, EPTP // 64, gbody, 0)

    _scatter_all(sp_t)

    plsc.subcore_barrier()

    pltpu.sync_copy(sp_t.at[pl.ds(nbase, NPT)], slice_buf)

    def fbody(i, c):
        for j in range(8):
            t = slice_buf[pl.ds(i * 128 + j * 16, 16)]
            dv = dinv_slice[pl.ds(i * 128 + j * 16, 16)]
            slice_buf[pl.ds(i * 128 + j * 16, 16)] = t * dv
        return c
    lax.fori_loop(0, NPT // 128, fbody, 0)
    pltpu.sync_copy(slice_buf, out_hbm.at[cid, pl.ds(nbase, NPT)])


def _mm_body(x_ref, win_ref, wout_ref, wloop_ref, a_in_ref, a_out_ref,
             a_loop_ref):
    x = x_ref[...]
    a_in_ref[...] = jnp.dot(
        x, win_ref[...], preferred_element_type=jnp.float32).astype(jnp.bfloat16)
    a_out_ref[...] = jnp.dot(
        x, wout_ref[...], preferred_element_type=jnp.float32).astype(jnp.bfloat16)
    a_loop_ref[...] = jnp.dot(
        x, wloop_ref[...], preferred_element_type=jnp.float32).astype(jnp.bfloat16)


NBLK = 5
BLK = N // NBLK


def _combine_body(a_in_hbm, a_out_hbm, a_loop_hbm, s_ref, g_ref, b_ref,
                  o_hbm, a1v, a2v, a3v, pv, sem_in2, sem_out):
    for k in range(NBLK):
        r = pl.ds(k * BLK, BLK)
        pltpu.async_copy(a_in_hbm.at[r], a1v.at[r], sem_in2)
        pltpu.async_copy(a_out_hbm.at[r], a2v.at[r], sem_in2)
        pltpu.async_copy(a_loop_hbm.at[r], a3v.at[r], sem_in2)

    sin = jnp.transpose(s_ref[0:1, :N])
    sout = jnp.transpose(s_ref[1:2, :N])

    acc = jnp.zeros((1, D), jnp.float32)
    acc2 = jnp.zeros((1, D), jnp.float32)
    for k in range(NBLK):
        r = pl.ds(k * BLK, BLK)
        pltpu.make_async_copy(a_in_hbm.at[r], a1v.at[r], sem_in2).wait()
        pltpu.make_async_copy(a_out_hbm.at[r], a2v.at[r], sem_in2).wait()
        pltpu.make_async_copy(a_loop_hbm.at[r], a3v.at[r], sem_in2).wait()
        pre = (a1v[r, :].astype(jnp.float32) * sin[k * BLK:(k + 1) * BLK, :]
               + a2v[r, :].astype(jnp.float32) * sout[k * BLK:(k + 1) * BLK, :]
               + a3v[r, :].astype(jnp.float32)) * jnp.float32(1.0 / 3.0)
        pv[r, :] = pre
        acc = acc + jnp.sum(pre, axis=0, keepdims=True)
        acc2 = acc2 + jnp.sum(pre * pre, axis=0, keepdims=True)

    mean = acc * jnp.float32(1.0 / N)
    var = acc2 * jnp.float32(1.0 / N) - mean * mean
    ginv = g_ref[...] * lax.rsqrt(var + 1e-5)
    shift = b_ref[...] - mean * ginv

    for k in range(NBLK):
        r = pl.ds(k * BLK, BLK)
        pv[r, :] = jnp.tanh(pv[r, :] * ginv + shift)
        pltpu.async_copy(pv.at[r], o_hbm.at[r], sem_out)

    for k in range(NBLK):
        r = pl.ds(k * BLK, BLK)
        pltpu.make_async_copy(pv.at[r], o_hbm.at[r], sem_out).wait()


def kernel(batch, x, edge_index, rel_embed, W_in, W_out, W_loop, gamma, beta):
    ei4 = edge_index.reshape(2, NCORE, NCHT, CHUNK)
    ei2 = edge_index.reshape(2, NCORE, EH)

    s2 = _sc_coeffs(ei4, ei2)

    bdt = jnp.bfloat16
    a_in, a_out, a_loop = pl.pallas_call(
        _mm_body,
        out_shape=(jax.ShapeDtypeStruct((N, D), bdt),
                   jax.ShapeDtypeStruct((N, D), bdt),
                   jax.ShapeDtypeStruct((N, D), bdt)),
    )(x, W_in, W_out, W_loop)

    out = pl.pallas_call(
        _combine_body,
        out_shape=jax.ShapeDtypeStruct((N, D), jnp.float32),
        in_specs=[
            pl.BlockSpec(memory_space=pl.ANY),
            pl.BlockSpec(memory_space=pl.ANY),
            pl.BlockSpec(memory_space=pl.ANY),
            pl.BlockSpec(memory_space=pltpu.VMEM),
            pl.BlockSpec(memory_space=pltpu.VMEM),
            pl.BlockSpec(memory_space=pltpu.VMEM),
        ],
        out_specs=pl.BlockSpec(memory_space=pl.ANY),
        scratch_shapes=[
            pltpu.VMEM((N, D), bdt),
            pltpu.VMEM((N, D), bdt),
            pltpu.VMEM((N, D), bdt),
            pltpu.VMEM((N, D), jnp.float32),
            pltpu.SemaphoreType.DMA,
            pltpu.SemaphoreType.DMA,
        ],
    )(a_in, a_out, a_loop, s2, gamma.reshape(1, D), beta.reshape(1, D))
    return out, rel_embed


# speedup vs baseline: 1.3992x; 1.0001x over previous
import functools

import jax
import jax.numpy as jnp
from jax import lax
from jax.experimental import pallas as pl
from jax.experimental.pallas import tpu as pltpu
from jax.experimental.pallas import tpu_sc as plsc

N = 10000
D = 128
E = 320000
EH = E // 2
NCORE = 2
NSUB = 16
EPT = EH // NSUB
CHUNK = 128
BCH = EPT // CHUNK
TAIL = EPT - BCH * CHUNK
EPTP = (BCH + 2) * CHUNK
NP = 10240
NPT = NP // NSUB
GRP = 26
NGRP = BCH // GRP


def _rsqrt_newton(d):
    seed = jnp.where(d < 64.0, 1.0 / 8.0,
                     jnp.where(d < 4096.0, 1.0 / 64.0, 1.0 / 512.0))
    y = seed.astype(jnp.float32)
    for _ in range(11):
        y = y * (1.5 - 0.5 * d * y * y)
    return jnp.where(d > 0.5, y, 0.0)


@functools.partial(
    pl.kernel,
    mesh=plsc.VectorSubcoreMesh(core_axis_name="c", subcore_axis_name="s"),
    out_type=jax.ShapeDtypeStruct((NCORE, NP), jnp.float32),
    compiler_params=pltpu.CompilerParams(needs_layout_passes=False),
    scratch_types=[
        pltpu.VMEM((2, EPTP), jnp.int32),
        pltpu.VMEM((EPT,), jnp.int32),
        pltpu.VMEM((EPT,), jnp.float32),
        pltpu.VMEM((NP,), jnp.float32),
        pltpu.VMEM((NPT,), jnp.float32),
        pltpu.VMEM((NPT,), jnp.float32),
        pltpu.VMEM_SHARED((NP,), jnp.float32),
        pltpu.VMEM_SHARED((NP,), jnp.float32),
        pltpu.VMEM_SHARED((NP,), jnp.float32),
        pltpu.SemaphoreType.DMA,
        pltpu.SemaphoreType.DMA,
    ],
)
def _sc_coeffs(ei_hbm, out_hbm, rowcol_buf, row_buf, vals_buf,
               dinv_full, slice_buf, dinv_slice, sp_deg, sp_t, sp_dinv,
               sem_in, sem_scat):
    cid = lax.axis_index("c")
    sid = lax.axis_index("s")
    nbase = sid * NPT
    ebase = cid * EH + sid * EPT
    abase = pl.multiple_of(
        jnp.minimum(ebase - (ebase % CHUNK), E - EPTP), CHUNK)
    ph = pl.multiple_of(ebase - abase, 16)

    pltpu.async_copy(ei_hbm.at[pl.ds(0, 2), pl.ds(abase, EPTP)], rowcol_buf,
                     sem_in)

    zeros = jnp.zeros((16,), jnp.float32)
    ones = jnp.ones((16,), jnp.float32)

    def zbody(i, c):
        for j in range(8):
            slice_buf[pl.ds(i * 128 + j * 16, 16)] = zeros
        return c
    lax.fori_loop(0, NPT // 128, zbody, 0)
    pltpu.sync_copy(slice_buf, sp_deg.at[pl.ds(nbase, NPT)])
    pltpu.sync_copy(slice_buf, sp_t.at[pl.ds(nbase, NPT)])

    def obody(i, c):
        for j in range(8):
            vals_buf[pl.ds(i * 128 + j * 16, 16)] = ones
        return c
    lax.fori_loop(0, EPT // 128, obody, 0)
    for j in range(TAIL // 16):
        vals_buf[pl.ds(BCH * CHUNK + j * 16, 16)] = ones

    pltpu.make_async_copy(ei_hbm.at[pl.ds(0, 2), pl.ds(abase, EPTP)],
                          rowcol_buf, sem_in).wait()

    def rbody(i, c):
        for j in range(5):
            row_buf[pl.ds(i * 80 + j * 16, 16)] = rowcol_buf[
                0, pl.ds(ph + i * 80 + j * 16, 16)]
        return c
    lax.fori_loop(0, EPT // 80, rbody, 0)

    plsc.subcore_barrier()

    def _scatter_all(target):
        def grp_body(g, c):
            def start_body(j, c2):
                k = g * GRP + j
                pltpu.async_copy(vals_buf.at[pl.ds(k * CHUNK, CHUNK)],
                                 target.at[row_buf.at[pl.ds(k * CHUNK, CHUNK)]],
                                 sem_scat, add=True)
                return c2
            lax.fori_loop(0, GRP, start_body, 0)

            def drain_body(j, c2):
                k = g * GRP + j
                pltpu.make_async_copy(
                    vals_buf.at[pl.ds(k * CHUNK, CHUNK)],
                    target.at[row_buf.at[pl.ds(k * CHUNK, CHUNK)]],
                    sem_scat).wait()
                return c2
            lax.fori_loop(0, GRP, drain_body, 0)
            return c
        lax.fori_loop(0, NGRP, grp_body, 0)

        pltpu.sync_copy(vals_buf.at[pl.ds(BCH * CHUNK, TAIL)],
                        target.at[row_buf.at[pl.ds(BCH * CHUNK, TAIL)]],
                        add=True)

    _scatter_all(sp_deg)

    plsc.subcore_barrier()

    pltpu.sync_copy(sp_deg.at[pl.ds(nbase, NPT)], slice_buf)

    def dbody(i, c):
        for j in range(2):
            d = slice_buf[pl.ds(i * 32 + j * 16, 16)]
            dinv_slice[pl.ds(i * 32 + j * 16, 16)] = _rsqrt_newton(d)
        return c
    lax.fori_loop(0, NPT // 32, dbody, 0)
    pltpu.sync_copy(dinv_slice, sp_dinv.at[pl.ds(nbase, NPT)])

    plsc.subcore_barrier()

    pltpu.sync_copy(sp_dinv, dinv_full)

    def gbody(i, c):
        for j in range(5):
            cidx = rowcol_buf[1, pl.ds(ph + i * 80 + j * 16, 16)]
            vals_buf[pl.ds(i * 80 + j * 16, 16)] = plsc.load_gather(
                dinv_full, [cidx])
        return c
    lax.fori_loop(0, EPT // 80, gbody, 0)

    _scatter_all(sp_t)

    plsc.subcore_barrier()

    pltpu.sync_copy(sp_t.at[pl.ds(nbase, NPT)], slice_buf)

    def fbody(i, c):
        for j in range(8):
            t = slice_buf[pl.ds(i * 128 + j * 16, 16)]
            dv = dinv_slice[pl.ds(i * 128 + j * 16, 16)]
            slice_buf[pl.ds(i * 128 + j * 16, 16)] = t * dv
        return c
    lax.fori_loop(0, NPT // 128, fbody, 0)
    pltpu.sync_copy(slice_buf, out_hbm.at[cid, pl.ds(nbase, NPT)])


def _mm_body(x_ref, win_ref, wout_ref, wloop_ref, a_in_ref, a_out_ref,
             a_loop_ref):
    x = x_ref[...]
    a_in_ref[...] = jnp.dot(
        x, win_ref[...], preferred_element_type=jnp.float32).astype(jnp.bfloat16)
    a_out_ref[...] = jnp.dot(
        x, wout_ref[...], preferred_element_type=jnp.float32).astype(jnp.bfloat16)
    a_loop_ref[...] = jnp.dot(
        x, wloop_ref[...], preferred_element_type=jnp.float32).astype(jnp.bfloat16)


NBLK = 5
BLK = N // NBLK


def _combine_body(a_in_hbm, a_out_hbm, a_loop_hbm, s_ref, g_ref, b_ref,
                  o_hbm, a1v, a2v, a3v, pv, sem_in2, sem_out):
    for k in range(NBLK):
        r = pl.ds(k * BLK, BLK)
        pltpu.async_copy(a_in_hbm.at[r], a1v.at[r], sem_in2)
        pltpu.async_copy(a_out_hbm.at[r], a2v.at[r], sem_in2)
        pltpu.async_copy(a_loop_hbm.at[r], a3v.at[r], sem_in2)

    sin = jnp.transpose(s_ref[0:1, :N])
    sout = jnp.transpose(s_ref[1:2, :N])

    acc = jnp.zeros((1, D), jnp.float32)
    acc2 = jnp.zeros((1, D), jnp.float32)
    for k in range(NBLK):
        r = pl.ds(k * BLK, BLK)
        pltpu.make_async_copy(a_in_hbm.at[r], a1v.at[r], sem_in2).wait()
        pltpu.make_async_copy(a_out_hbm.at[r], a2v.at[r], sem_in2).wait()
        pltpu.make_async_copy(a_loop_hbm.at[r], a3v.at[r], sem_in2).wait()
        pre = (a1v[r, :].astype(jnp.float32) * sin[k * BLK:(k + 1) * BLK, :]
               + a2v[r, :].astype(jnp.float32) * sout[k * BLK:(k + 1) * BLK, :]
               + a3v[r, :].astype(jnp.float32)) * jnp.float32(1.0 / 3.0)
        pv[r, :] = pre
        acc = acc + jnp.sum(pre, axis=0, keepdims=True)
        acc2 = acc2 + jnp.sum(pre * pre, axis=0, keepdims=True)

    mean = acc * jnp.float32(1.0 / N)
    var = acc2 * jnp.float32(1.0 / N) - mean * mean
    ginv = g_ref[...] * lax.rsqrt(var + 1e-5)
    shift = b_ref[...] - mean * ginv

    for k in range(NBLK):
        r = pl.ds(k * BLK, BLK)
        pv[r, :] = jnp.tanh(pv[r, :] * ginv + shift)
        pltpu.async_copy(pv.at[r], o_hbm.at[r], sem_out)

    for k in range(NBLK):
        r = pl.ds(k * BLK, BLK)
        pltpu.make_async_copy(pv.at[r], o_hbm.at[r], sem_out).wait()


def kernel(batch, x, edge_index, rel_embed, W_in, W_out, W_loop, gamma, beta):
    s2 = _sc_coeffs(edge_index)

    bdt = jnp.bfloat16
    a_in, a_out, a_loop = pl.pallas_call(
        _mm_body,
        out_shape=(jax.ShapeDtypeStruct((N, D), bdt),
                   jax.ShapeDtypeStruct((N, D), bdt),
                   jax.ShapeDtypeStruct((N, D), bdt)),
    )(x, W_in, W_out, W_loop)

    out = pl.pallas_call(
        _combine_body,
        out_shape=jax.ShapeDtypeStruct((N, D), jnp.float32),
        in_specs=[
            pl.BlockSpec(memory_space=pl.ANY),
            pl.BlockSpec(memory_space=pl.ANY),
            pl.BlockSpec(memory_space=pl.ANY),
            pl.BlockSpec(memory_space=pltpu.VMEM),
            pl.BlockSpec(memory_space=pltpu.VMEM),
            pl.BlockSpec(memory_space=pltpu.VMEM),
        ],
        out_specs=pl.BlockSpec(memory_space=pl.ANY),
        scratch_shapes=[
            pltpu.VMEM((N, D), bdt),
            pltpu.VMEM((N, D), bdt),
            pltpu.VMEM((N, D), bdt),
            pltpu.VMEM((N, D), jnp.float32),
            pltpu.SemaphoreType.DMA,
            pltpu.SemaphoreType.DMA,
        ],
    )(a_in, a_out, a_loop, s2, gamma.reshape(1, D), beta.reshape(1, D))
    return out, rel_embed


# speedup vs baseline: 1.4333x; 1.0244x over previous
import functools

import jax
import jax.numpy as jnp
from jax import lax
from jax.experimental import pallas as pl
from jax.experimental.pallas import tpu as pltpu
from jax.experimental.pallas import tpu_sc as plsc

N = 10000
D = 128
E = 320000
EH = E // 2
NCORE = 2
NSUB = 16
EPT = EH // NSUB
CHUNK = 128
BCH = EPT // CHUNK
TAIL = EPT - BCH * CHUNK
EPTP = (BCH + 2) * CHUNK
NP = 10240
NPT = NP // NSUB
GRP = 26
NGRP = BCH // GRP


def _rsqrt_newton(d):
    seed = jnp.where(d < 64.0, 1.0 / 8.0,
                     jnp.where(d < 4096.0, 1.0 / 64.0, 1.0 / 512.0))
    y = seed.astype(jnp.float32)
    for _ in range(11):
        y = y * (1.5 - 0.5 * d * y * y)
    return jnp.where(d > 0.5, y, 0.0)


@functools.partial(
    pl.kernel,
    mesh=plsc.VectorSubcoreMesh(core_axis_name="c", subcore_axis_name="s"),
    out_type=jax.ShapeDtypeStruct((NCORE, NP), jnp.float32),
    compiler_params=pltpu.CompilerParams(needs_layout_passes=False),
    scratch_types=[
        pltpu.VMEM((2, EPTP), jnp.int32),
        pltpu.VMEM((EPT,), jnp.int32),
        pltpu.VMEM((EPT,), jnp.int32),
        pltpu.VMEM((EPT,), jnp.float32),
        pltpu.VMEM((NP,), jnp.float32),
        pltpu.VMEM((NPT,), jnp.float32),
        pltpu.VMEM((NPT,), jnp.float32),
        pltpu.VMEM_SHARED((NP,), jnp.float32),
        pltpu.VMEM_SHARED((NP,), jnp.float32),
        pltpu.VMEM_SHARED((NP,), jnp.float32),
        pltpu.SemaphoreType.DMA,
        pltpu.SemaphoreType.DMA,
    ],
)
def _sc_coeffs(ei_hbm, out_hbm, rowcol_buf, row_buf, col_buf, vals_buf,
               dinv_full, slice_buf, dinv_slice, sp_deg, sp_t, sp_dinv,
               sem_in, sem_scat):
    cid = lax.axis_index("c")
    sid = lax.axis_index("s")
    nbase = sid * NPT
    ebase = cid * EH + sid * EPT
    abase = pl.multiple_of(
        jnp.minimum(ebase - (ebase % CHUNK), E - EPTP), CHUNK)
    ph = pl.multiple_of(ebase - abase, 16)

    pltpu.async_copy(ei_hbm.at[pl.ds(0, 2), pl.ds(abase, EPTP)], rowcol_buf,
                     sem_in)

    zeros = jnp.zeros((16,), jnp.float32)
    ones = jnp.ones((16,), jnp.float32)

    def zbody(i, c):
        for j in range(8):
            slice_buf[pl.ds(i * 128 + j * 16, 16)] = zeros
        return c
    lax.fori_loop(0, NPT // 128, zbody, 0)
    pltpu.sync_copy(slice_buf, sp_deg.at[pl.ds(nbase, NPT)])
    pltpu.sync_copy(slice_buf, sp_t.at[pl.ds(nbase, NPT)])

    def obody(i, c):
        for j in range(8):
            vals_buf[pl.ds(i * 128 + j * 16, 16)] = ones
        return c
    lax.fori_loop(0, EPT // 128, obody, 0)
    for j in range(TAIL // 16):
        vals_buf[pl.ds(BCH * CHUNK + j * 16, 16)] = ones

    pltpu.make_async_copy(ei_hbm.at[pl.ds(0, 2), pl.ds(abase, EPTP)],
                          rowcol_buf, sem_in).wait()

    def rbody(i, c):
        for j in range(5):
            row_buf[pl.ds(i * 80 + j * 16, 16)] = rowcol_buf[
                0, pl.ds(ph + i * 80 + j * 16, 16)]
            col_buf[pl.ds(i * 80 + j * 16, 16)] = rowcol_buf[
                1, pl.ds(ph + i * 80 + j * 16, 16)]
        return c
    lax.fori_loop(0, EPT // 80, rbody, 0)

    plsc.subcore_barrier()

    def _scatter_all(target):
        def grp_body(g, c):
            def start_body(j, c2):
                k = g * GRP + j
                pltpu.async_copy(vals_buf.at[pl.ds(k * CHUNK, CHUNK)],
                                 target.at[row_buf.at[pl.ds(k * CHUNK, CHUNK)]],
                                 sem_scat, add=True)
                return c2
            lax.fori_loop(0, GRP, start_body, 0)

            def drain_body(j, c2):
                k = g * GRP + j
                pltpu.make_async_copy(
                    vals_buf.at[pl.ds(k * CHUNK, CHUNK)],
                    target.at[row_buf.at[pl.ds(k * CHUNK, CHUNK)]],
                    sem_scat).wait()
                return c2
            lax.fori_loop(0, GRP, drain_body, 0)
            return c
        lax.fori_loop(0, NGRP, grp_body, 0)

        pltpu.sync_copy(vals_buf.at[pl.ds(BCH * CHUNK, TAIL)],
                        target.at[row_buf.at[pl.ds(BCH * CHUNK, TAIL)]],
                        add=True)

    _scatter_all(sp_deg)

    plsc.subcore_barrier()

    pltpu.sync_copy(sp_deg.at[pl.ds(nbase, NPT)], slice_buf)

    def dbody(i, c):
        for j in range(2):
            d = slice_buf[pl.ds(i * 32 + j * 16, 16)]
            dinv_slice[pl.ds(i * 32 + j * 16, 16)] = _rsqrt_newton(d)
        return c
    lax.fori_loop(0, NPT // 32, dbody, 0)
    pltpu.sync_copy(dinv_slice, sp_dinv.at[pl.ds(nbase, NPT)])

    plsc.subcore_barrier()

    pltpu.sync_copy(sp_dinv, dinv_full)

    def gbody(i, c):
        for j in range(5):
            cidx = col_buf[pl.ds(i * 80 + j * 16, 16)]
            vals_buf[pl.ds(i * 80 + j * 16, 16)] = plsc.load_gather(
                dinv_full, [cidx])
        return c
    lax.fori_loop(0, EPT // 80, gbody, 0)

    _scatter_all(sp_t)

    plsc.subcore_barrier()

    pltpu.sync_copy(sp_t.at[pl.ds(nbase, NPT)], slice_buf)

    def fbody(i, c):
        for j in range(8):
            t = slice_buf[pl.ds(i * 128 + j * 16, 16)]
            dv = dinv_slice[pl.ds(i * 128 + j * 16, 16)]
            slice_buf[pl.ds(i * 128 + j * 16, 16)] = t * dv
        return c
    lax.fori_loop(0, NPT // 128, fbody, 0)
    pltpu.sync_copy(slice_buf, out_hbm.at[cid, pl.ds(nbase, NPT)])


def _mm_body(x_ref, win_ref, wout_ref, wloop_ref, a_in_ref, a_out_ref,
             a_loop_ref):
    x = x_ref[...]
    a_in_ref[...] = jnp.dot(
        x, win_ref[...], preferred_element_type=jnp.float32).astype(jnp.bfloat16)
    a_out_ref[...] = jnp.dot(
        x, wout_ref[...], preferred_element_type=jnp.float32).astype(jnp.bfloat16)
    a_loop_ref[...] = jnp.dot(
        x, wloop_ref[...], preferred_element_type=jnp.float32).astype(jnp.bfloat16)


NBLK = 5
BLK = N // NBLK


def _combine_body(a_in_hbm, a_out_hbm, a_loop_hbm, s_ref, g_ref, b_ref,
                  o_hbm, a1v, a2v, a3v, pv, sem_in2, sem_out):
    for k in range(NBLK):
        r = pl.ds(k * BLK, BLK)
        pltpu.async_copy(a_in_hbm.at[r], a1v.at[r], sem_in2)
        pltpu.async_copy(a_out_hbm.at[r], a2v.at[r], sem_in2)
        pltpu.async_copy(a_loop_hbm.at[r], a3v.at[r], sem_in2)

    sin = jnp.transpose(s_ref[0:1, :N])
    sout = jnp.transpose(s_ref[1:2, :N])

    acc = jnp.zeros((1, D), jnp.float32)
    acc2 = jnp.zeros((1, D), jnp.float32)
    for k in range(NBLK):
        r = pl.ds(k * BLK, BLK)
        pltpu.make_async_copy(a_in_hbm.at[r], a1v.at[r], sem_in2).wait()
        pltpu.make_async_copy(a_out_hbm.at[r], a2v.at[r], sem_in2).wait()
        pltpu.make_async_copy(a_loop_hbm.at[r], a3v.at[r], sem_in2).wait()
        pre = (a1v[r, :].astype(jnp.float32) * sin[k * BLK:(k + 1) * BLK, :]
               + a2v[r, :].astype(jnp.float32) * sout[k * BLK:(k + 1) * BLK, :]
               + a3v[r, :].astype(jnp.float32)) * jnp.float32(1.0 / 3.0)
        pv[r, :] = pre
        acc = acc + jnp.sum(pre, axis=0, keepdims=True)
        acc2 = acc2 + jnp.sum(pre * pre, axis=0, keepdims=True)

    mean = acc * jnp.float32(1.0 / N)
    var = acc2 * jnp.float32(1.0 / N) - mean * mean
    ginv = g_ref[...] * lax.rsqrt(var + 1e-5)
    shift = b_ref[...] - mean * ginv

    for k in range(NBLK):
        r = pl.ds(k * BLK, BLK)
        pv[r, :] = jnp.tanh(pv[r, :] * ginv + shift)
        pltpu.async_copy(pv.at[r], o_hbm.at[r], sem_out)

    for k in range(NBLK):
        r = pl.ds(k * BLK, BLK)
        pltpu.make_async_copy(pv.at[r], o_hbm.at[r], sem_out).wait()


def kernel(batch, x, edge_index, rel_embed, W_in, W_out, W_loop, gamma, beta):
    s2 = _sc_coeffs(edge_index)

    bdt = jnp.bfloat16
    a_in, a_out, a_loop = pl.pallas_call(
        _mm_body,
        out_shape=(jax.ShapeDtypeStruct((N, D), bdt),
                   jax.ShapeDtypeStruct((N, D), bdt),
                   jax.ShapeDtypeStruct((N, D), bdt)),
    )(x, W_in, W_out, W_loop)

    out = pl.pallas_call(
        _combine_body,
        out_shape=jax.ShapeDtypeStruct((N, D), jnp.float32),
        in_specs=[
            pl.BlockSpec(memory_space=pl.ANY),
            pl.BlockSpec(memory_space=pl.ANY),
            pl.BlockSpec(memory_space=pl.ANY),
            pl.BlockSpec(memory_space=pltpu.VMEM),
            pl.BlockSpec(memory_space=pltpu.VMEM),
            pl.BlockSpec(memory_space=pltpu.VMEM),
        ],
        out_specs=pl.BlockSpec(memory_space=pl.ANY),
        scratch_shapes=[
            pltpu.VMEM((N, D), bdt),
            pltpu.VMEM((N, D), bdt),
            pltpu.VMEM((N, D), bdt),
            pltpu.VMEM((N, D), jnp.float32),
            pltpu.SemaphoreType.DMA,
            pltpu.SemaphoreType.DMA,
        ],
    )(a_in, a_out, a_loop, s2, gamma.reshape(1, D), beta.reshape(1, D))
    return out, rel_embed


# speedup vs baseline: 1.4346x; 1.0008x over previous
import functools

import jax
import jax.numpy as jnp
from jax import lax
from jax.experimental import pallas as pl
from jax.experimental.pallas import tpu as pltpu
from jax.experimental.pallas import tpu_sc as plsc

N = 10000
D = 128
E = 320000
EH = E // 2
NCORE = 2
NSUB = 16
EPT = EH // NSUB
CHUNK = 128
BCH = EPT // CHUNK
TAIL = EPT - BCH * CHUNK
EPTP = (BCH + 2) * CHUNK
NP = 10240
NPT = NP // NSUB
GRP = 39
NGRP = BCH // GRP


def _rsqrt_newton(d):
    seed = jnp.where(d < 64.0, 1.0 / 8.0,
                     jnp.where(d < 4096.0, 1.0 / 64.0, 1.0 / 512.0))
    y = seed.astype(jnp.float32)
    for _ in range(11):
        y = y * (1.5 - 0.5 * d * y * y)
    return jnp.where(d > 0.5, y, 0.0)


@functools.partial(
    pl.kernel,
    mesh=plsc.VectorSubcoreMesh(core_axis_name="c", subcore_axis_name="s"),
    out_type=jax.ShapeDtypeStruct((NCORE, NP), jnp.float32),
    compiler_params=pltpu.CompilerParams(needs_layout_passes=False),
    scratch_types=[
        pltpu.VMEM((2, EPTP), jnp.int32),
        pltpu.VMEM((EPT,), jnp.int32),
        pltpu.VMEM((EPT,), jnp.int32),
        pltpu.VMEM((EPT,), jnp.float32),
        pltpu.VMEM((NP,), jnp.float32),
        pltpu.VMEM((NPT,), jnp.float32),
        pltpu.VMEM((NPT,), jnp.float32),
        pltpu.VMEM_SHARED((NP,), jnp.float32),
        pltpu.VMEM_SHARED((NP,), jnp.float32),
        pltpu.VMEM_SHARED((NP,), jnp.float32),
        pltpu.SemaphoreType.DMA,
        pltpu.SemaphoreType.DMA,
    ],
)
def _sc_coeffs(ei_hbm, out_hbm, rowcol_buf, row_buf, col_buf, vals_buf,
               dinv_full, slice_buf, dinv_slice, sp_deg, sp_t, sp_dinv,
               sem_in, sem_scat):
    cid = lax.axis_index("c")
    sid = lax.axis_index("s")
    nbase = sid * NPT
    ebase = cid * EH + sid * EPT
    abase = pl.multiple_of(
        jnp.minimum(ebase - (ebase % CHUNK), E - EPTP), CHUNK)
    ph = pl.multiple_of(ebase - abase, 16)

    pltpu.async_copy(ei_hbm.at[pl.ds(0, 2), pl.ds(abase, EPTP)], rowcol_buf,
                     sem_in)

    zeros = jnp.zeros((16,), jnp.float32)
    ones = jnp.ones((16,), jnp.float32)

    def zbody(i, c):
        for j in range(8):
            slice_buf[pl.ds(i * 128 + j * 16, 16)] = zeros
        return c
    lax.fori_loop(0, NPT // 128, zbody, 0)
    pltpu.sync_copy(slice_buf, sp_deg.at[pl.ds(nbase, NPT)])
    pltpu.sync_copy(slice_buf, sp_t.at[pl.ds(nbase, NPT)])

    def obody(i, c):
        for j in range(8):
            vals_buf[pl.ds(i * 128 + j * 16, 16)] = ones
        return c
    lax.fori_loop(0, EPT // 128, obody, 0)
    for j in range(TAIL // 16):
        vals_buf[pl.ds(BCH * CHUNK + j * 16, 16)] = ones

    pltpu.make_async_copy(ei_hbm.at[pl.ds(0, 2), pl.ds(abase, EPTP)],
                          rowcol_buf, sem_in).wait()

    def rbody(i, c):
        for j in range(5):
            row_buf[pl.ds(i * 80 + j * 16, 16)] = rowcol_buf[
                0, pl.ds(ph + i * 80 + j * 16, 16)]
            col_buf[pl.ds(i * 80 + j * 16, 16)] = rowcol_buf[
                1, pl.ds(ph + i * 80 + j * 16, 16)]
        return c
    lax.fori_loop(0, EPT // 80, rbody, 0)

    plsc.subcore_barrier()

    def _scatter_all(target):
        def grp_body(g, c):
            def start_body(j, c2):
                k = g * GRP + j
                pltpu.async_copy(vals_buf.at[pl.ds(k * CHUNK, CHUNK)],
                                 target.at[row_buf.at[pl.ds(k * CHUNK, CHUNK)]],
                                 sem_scat, add=True)
                return c2
            lax.fori_loop(0, GRP, start_body, 0)

            def drain_body(j, c2):
                k = g * GRP + j
                pltpu.make_async_copy(
                    vals_buf.at[pl.ds(k * CHUNK, CHUNK)],
                    target.at[row_buf.at[pl.ds(k * CHUNK, CHUNK)]],
                    sem_scat).wait()
                return c2
            lax.fori_loop(0, GRP, drain_body, 0)
            return c
        lax.fori_loop(0, NGRP, grp_body, 0)

        pltpu.sync_copy(vals_buf.at[pl.ds(BCH * CHUNK, TAIL)],
                        target.at[row_buf.at[pl.ds(BCH * CHUNK, TAIL)]],
                        add=True)

    _scatter_all(sp_deg)

    plsc.subcore_barrier()

    pltpu.sync_copy(sp_deg.at[pl.ds(nbase, NPT)], slice_buf)

    def dbody(i, c):
        for j in range(2):
            d = slice_buf[pl.ds(i * 32 + j * 16, 16)]
            dinv_slice[pl.ds(i * 32 + j * 16, 16)] = _rsqrt_newton(d)
        return c
    lax.fori_loop(0, NPT // 32, dbody, 0)
    pltpu.sync_copy(dinv_slice, sp_dinv.at[pl.ds(nbase, NPT)])

    plsc.subcore_barrier()

    pltpu.sync_copy(sp_dinv, dinv_full)

    def gbody(i, c):
        for j in range(5):
            cidx = col_buf[pl.ds(i * 80 + j * 16, 16)]
            vals_buf[pl.ds(i * 80 + j * 16, 16)] = plsc.load_gather(
                dinv_full, [cidx])
        return c
    lax.fori_loop(0, EPT // 80, gbody, 0)

    _scatter_all(sp_t)

    plsc.subcore_barrier()

    pltpu.sync_copy(sp_t.at[pl.ds(nbase, NPT)], slice_buf)

    def fbody(i, c):
        for j in range(8):
            t = slice_buf[pl.ds(i * 128 + j * 16, 16)]
            dv = dinv_slice[pl.ds(i * 128 + j * 16, 16)]
            slice_buf[pl.ds(i * 128 + j * 16, 16)] = t * dv
        return c
    lax.fori_loop(0, NPT // 128, fbody, 0)
    pltpu.sync_copy(slice_buf, out_hbm.at[cid, pl.ds(nbase, NPT)])


def _mm_body(x_ref, win_ref, wout_ref, wloop_ref, a_in_ref, a_out_ref,
             a_loop_ref):
    x = x_ref[...]
    a_in_ref[...] = jnp.dot(
        x, win_ref[...], preferred_element_type=jnp.float32).astype(jnp.bfloat16)
    a_out_ref[...] = jnp.dot(
        x, wout_ref[...], preferred_element_type=jnp.float32).astype(jnp.bfloat16)
    a_loop_ref[...] = jnp.dot(
        x, wloop_ref[...], preferred_element_type=jnp.float32).astype(jnp.bfloat16)


NBLK = 5
BLK = N // NBLK


def _combine_body(a_in_hbm, a_out_hbm, a_loop_hbm, s_ref, g_ref, b_ref,
                  o_hbm, a1v, a2v, a3v, pv, sem_in2, sem_out):
    for k in range(NBLK):
        r = pl.ds(k * BLK, BLK)
        pltpu.async_copy(a_in_hbm.at[r], a1v.at[r], sem_in2)
        pltpu.async_copy(a_out_hbm.at[r], a2v.at[r], sem_in2)
        pltpu.async_copy(a_loop_hbm.at[r], a3v.at[r], sem_in2)

    sin = jnp.transpose(s_ref[0:1, :N])
    sout = jnp.transpose(s_ref[1:2, :N])

    acc = jnp.zeros((1, D), jnp.float32)
    acc2 = jnp.zeros((1, D), jnp.float32)
    for k in range(NBLK):
        r = pl.ds(k * BLK, BLK)
        pltpu.make_async_copy(a_in_hbm.at[r], a1v.at[r], sem_in2).wait()
        pltpu.make_async_copy(a_out_hbm.at[r], a2v.at[r], sem_in2).wait()
        pltpu.make_async_copy(a_loop_hbm.at[r], a3v.at[r], sem_in2).wait()
        pre = (a1v[r, :].astype(jnp.float32) * sin[k * BLK:(k + 1) * BLK, :]
               + a2v[r, :].astype(jnp.float32) * sout[k * BLK:(k + 1) * BLK, :]
               + a3v[r, :].astype(jnp.float32)) * jnp.float32(1.0 / 3.0)
        pv[r, :] = pre
        acc = acc + jnp.sum(pre, axis=0, keepdims=True)
        acc2 = acc2 + jnp.sum(pre * pre, axis=0, keepdims=True)

    mean = acc * jnp.float32(1.0 / N)
    var = acc2 * jnp.float32(1.0 / N) - mean * mean
    ginv = g_ref[...] * lax.rsqrt(var + 1e-5)
    shift = b_ref[...] - mean * ginv

    for k in range(NBLK):
        r = pl.ds(k * BLK, BLK)
        pv[r, :] = jnp.tanh(pv[r, :] * ginv + shift)
        pltpu.async_copy(pv.at[r], o_hbm.at[r], sem_out)

    for k in range(NBLK):
        r = pl.ds(k * BLK, BLK)
        pltpu.make_async_copy(pv.at[r], o_hbm.at[r], sem_out).wait()


def kernel(batch, x, edge_index, rel_embed, W_in, W_out, W_loop, gamma, beta):
    s2 = _sc_coeffs(edge_index)

    bdt = jnp.bfloat16
    a_in, a_out, a_loop = pl.pallas_call(
        _mm_body,
        out_shape=(jax.ShapeDtypeStruct((N, D), bdt),
                   jax.ShapeDtypeStruct((N, D), bdt),
                   jax.ShapeDtypeStruct((N, D), bdt)),
    )(x, W_in, W_out, W_loop)

    out = pl.pallas_call(
        _combine_body,
        out_shape=jax.ShapeDtypeStruct((N, D), jnp.float32),
        in_specs=[
            pl.BlockSpec(memory_space=pl.ANY),
            pl.BlockSpec(memory_space=pl.ANY),
            pl.BlockSpec(memory_space=pl.ANY),
            pl.BlockSpec(memory_space=pltpu.VMEM),
            pl.BlockSpec(memory_space=pltpu.VMEM),
            pl.BlockSpec(memory_space=pltpu.VMEM),
        ],
        out_specs=pl.BlockSpec(memory_space=pl.ANY),
        scratch_shapes=[
            pltpu.VMEM((N, D), bdt),
            pltpu.VMEM((N, D), bdt),
            pltpu.VMEM((N, D), bdt),
            pltpu.VMEM((N, D), jnp.float32),
            pltpu.SemaphoreType.DMA,
            pltpu.SemaphoreType.DMA,
        ],
    )(a_in, a_out, a_loop, s2, gamma.reshape(1, D), beta.reshape(1, D))
    return out, rel_embed
